# Initial kernel scaffold; baseline (speedup 1.0000x reference)
import jax
import jax.numpy as jnp
from jax.experimental import pallas as pl


def kernel(features, enabled, w1, b1, w2, b2, w3, b3):
    raise NotImplementedError("write your pallas kernel here")


# speedup vs baseline: 25.1699x; 25.1699x over previous
import jax
import jax.numpy as jnp
import numpy as np
from jax.experimental import pallas as pl

_B, _C, _H, _W = 8, 384, 32, 32
_MID, _GROUPS = 96, 8
_P = _H * _W
_KEEP = 256
_PAD = 64
_INT_MIN = np.int32(-2147483648)
_SQRT1_2 = np.float32(1.0 / np.sqrt(2.0))


def _gelu(x):
    return 0.5 * x * (1.0 + jax.lax.erf(x * _SQRT1_2))


def _body(x_ref, w1_ref, b1_ref, w2_ref, b2_ref, w3_ref, b3_ref,
          mask_ref, scores_ref):
    X = x_ref[0]
    H1 = _gelu(jnp.dot(w1_ref[...], X, preferred_element_type=jnp.float32)
               + b1_ref[...])
    zpad = jnp.zeros((_MID, _PAD), jnp.float32)
    H1p = jnp.concatenate([zpad, H1, zpad], axis=1)
    xcol = jax.lax.broadcasted_iota(jnp.int32, (1, _P), 1) % _W
    acc = b2_ref[...] + jnp.zeros((_MID, _P), jnp.float32)
    for dy in (-1, 0, 1):
        for dx in (-1, 0, 1):
            s = dy * _W + dx
            sh = jax.lax.slice(H1p, (0, _PAD + s), (_MID, _PAD + s + _P))
            if dx == 1:
                sh = jnp.where(xcol != (_W - 1), sh, 0.0)
            elif dx == -1:
                sh = jnp.where(xcol != 0, sh, 0.0)
            t = (dy + 1) * 3 + (dx + 1)
            acc = acc + jnp.dot(w2_ref[t], sh,
                                preferred_element_type=jnp.float32)
    H2 = _gelu(acc)
    S = jnp.dot(w3_ref[...], H2, preferred_element_type=jnp.float32) \
        + b3_ref[...]
    scores_ref[0] = S

    bits = jax.lax.bitcast_convert_type(S, jnp.int32)
    keys = jnp.where(bits >= 0, bits, bits ^ np.int32(0x7FFFFFFF))

    def step(i, T):
        cand_u = T | (np.int32(1) << (31 - i))
        cand_k = cand_u ^ _INT_MIN
        cnt = jnp.sum((keys < cand_k).astype(jnp.int32), axis=1,
                      keepdims=True)
        return jnp.where(cnt < _KEEP, cand_u, T)

    T = jax.lax.fori_loop(0, 32, step, jnp.zeros((_C, 1), jnp.int32))
    Tk = T ^ _INT_MIN
    lt = keys < Tk
    eq = keys == Tk
    g = jnp.sum(lt.astype(jnp.int32), axis=1, keepdims=True)
    c = eq.astype(jnp.int32)
    shift = 1
    while shift < _P:
        c = c + jnp.concatenate(
            [jnp.zeros((_C, shift), jnp.int32),
             jax.lax.slice(c, (0, 0), (_C, _P - shift))], axis=1)
        shift *= 2
    mask = lt | (eq & (c <= (_KEEP - g)))
    mask_ref[0] = mask.astype(jnp.float32)


def _block_diag_w2(w2):
    cpg = _MID // _GROUPS
    take = jnp.take(w2, jnp.asarray(np.arange(_MID) % cpg), axis=1)
    gi = np.arange(_MID) // cpg
    gmask = jnp.asarray((gi[:, None] == gi[None, :]).astype(np.float32))
    full = take * gmask[:, :, None, None]
    return jnp.transpose(full, (2, 3, 0, 1)).reshape(9, _MID, _MID)


def _run(xf, w1, b1, w2bd, b2, w3, b3):
    return pl.pallas_call(
        _body,
        grid=(_B,),
        in_specs=[
            pl.BlockSpec((1, _C, _P), lambda b: (b, 0, 0)),
            pl.BlockSpec((_MID, _C), lambda b: (0, 0)),
            pl.BlockSpec((_MID, 1), lambda b: (0, 0)),
            pl.BlockSpec((9, _MID, _MID), lambda b: (0, 0, 0)),
            pl.BlockSpec((_MID, 1), lambda b: (0, 0)),
            pl.BlockSpec((_C, _MID), lambda b: (0, 0)),
            pl.BlockSpec((_C, 1), lambda b: (0, 0)),
        ],
        out_specs=[
            pl.BlockSpec((1, _C, _P), lambda b: (b, 0, 0)),
            pl.BlockSpec((1, _C, _P), lambda b: (b, 0, 0)),
        ],
        out_shape=[
            jax.ShapeDtypeStruct((_B, _C, _P), jnp.float32),
            jax.ShapeDtypeStruct((_B, _C, _P), jnp.float32),
        ],
    )(xf, w1, b1, w2bd, b2, w3, b3)


def kernel(features, enabled, w1, b1, w2, b2, w3, b3):
    xf = features.reshape(_B, _C, _P)
    mask, scores = _run(
        xf,
        w1.reshape(_MID, _C),
        b1.reshape(_MID, 1),
        _block_diag_w2(w2),
        b2.reshape(_MID, 1),
        w3.reshape(_C, _MID),
        b3.reshape(_C, 1),
    )
    mask4 = mask.reshape(_B, _C, _H, _W)
    scores4 = scores.reshape(_B, _C, _H, _W)
    en = jnp.asarray(enabled) != 0
    out_mask = jnp.where(en, mask4, jnp.float32(1.0))
    out_scores = jnp.where(en, scores4, jnp.float32(0.0))
    return out_mask, out_scores


# speedup vs baseline: 34.3426x; 1.3644x over previous
import jax
import jax.numpy as jnp
import numpy as np
from jax.experimental import pallas as pl

_B, _C, _H, _W = 8, 384, 32, 32
_MID, _GROUPS = 96, 8
_P = _H * _W
_KEEP = 256
_PAD = 64
_INT_MIN = np.int32(-2147483648)
_SQRT1_2 = np.float32(1.0 / np.sqrt(2.0))


def _gelu(x):
    return 0.5 * x * (1.0 + jax.lax.erf(x * _SQRT1_2))


def _dot_t(a, b):
    return jax.lax.dot_general(a, b, (((1,), (1,)), ((), ())),
                               preferred_element_type=jnp.float32)


def _body(x_ref, w1_ref, b1_ref, w2_ref, b2_ref, w3_ref, b3_ref,
          mask_ref, scores_ref):
    X = x_ref[0]
    H1 = _gelu(jax.lax.dot_general(X, w1_ref[...], (((0,), (1,)), ((), ())),
                                   preferred_element_type=jnp.float32)
               + b1_ref[...])
    zpad = jnp.zeros((_PAD, _MID), jnp.float32)
    H1p = jnp.concatenate([zpad, H1, zpad], axis=0)
    xrow = jax.lax.broadcasted_iota(jnp.int32, (_P, 1), 0) % _W
    acc = b2_ref[...] + jnp.zeros((_P, _MID), jnp.float32)
    for dy in (-1, 0, 1):
        for dx in (-1, 0, 1):
            s = dy * _W + dx
            sh = jax.lax.slice(H1p, (_PAD + s, 0), (_PAD + s + _P, _MID))
            if dx == 1:
                sh = jnp.where(xrow != (_W - 1), sh, 0.0)
            elif dx == -1:
                sh = jnp.where(xrow != 0, sh, 0.0)
            t = (dy + 1) * 3 + (dx + 1)
            acc = acc + _dot_t(sh, w2_ref[t])
    H2 = _gelu(acc)
    S = _dot_t(H2, w3_ref[...]) + b3_ref[...]
    scores_ref[0] = jnp.transpose(S, (1, 0))

    bits = jax.lax.bitcast_convert_type(S, jnp.int32)
    keys = jnp.where(bits >= 0, bits, bits ^ np.int32(0x7FFFFFFF))

    def step(i, T):
        cand_u = T | (np.int32(1) << (31 - i))
        cand_k = cand_u ^ _INT_MIN
        cnt = jnp.sum((keys < cand_k).astype(jnp.int32), axis=0,
                      keepdims=True)
        return jnp.where(cnt < _KEEP, cand_u, T)

    T = jax.lax.fori_loop(0, 32, step, jnp.zeros((1, _C), jnp.int32))
    Tk = T ^ _INT_MIN
    le = keys <= Tk
    cnt_le = jnp.sum(le.astype(jnp.int32), axis=0, keepdims=True)
    any_tie = jnp.sum(jnp.where(cnt_le > _KEEP, 1, 0)) > 0

    def no_tie():
        return le.astype(jnp.float32)

    def tie_fix():
        lt = keys < Tk
        eq = keys == Tk
        g = jnp.sum(lt.astype(jnp.int32), axis=0, keepdims=True)
        c = eq.astype(jnp.int32)
        shift = 1
        while shift < _P:
            c = c + jnp.concatenate(
                [jnp.zeros((shift, _C), jnp.int32),
                 jax.lax.slice(c, (0, 0), (_P - shift, _C))], axis=0)
            shift *= 2
        return (lt | (eq & (c <= (_KEEP - g)))).astype(jnp.float32)

    maskT = jax.lax.cond(any_tie, tie_fix, no_tie)
    mask_ref[0] = jnp.transpose(maskT, (1, 0))


def _block_diag_w2(w2):
    cpg = _MID // _GROUPS
    take = jnp.take(w2, jnp.asarray(np.arange(_MID) % cpg), axis=1)
    gi = np.arange(_MID) // cpg
    gmask = jnp.asarray((gi[:, None] == gi[None, :]).astype(np.float32))
    full = take * gmask[:, :, None, None]
    return jnp.transpose(full, (2, 3, 0, 1)).reshape(9, _MID, _MID)


def _run(xf, w1, b1, w2bd, b2, w3, b3):
    return pl.pallas_call(
        _body,
        grid=(_B,),
        in_specs=[
            pl.BlockSpec((1, _C, _P), lambda b: (b, 0, 0)),
            pl.BlockSpec((_MID, _C), lambda b: (0, 0)),
            pl.BlockSpec((1, _MID), lambda b: (0, 0)),
            pl.BlockSpec((9, _MID, _MID), lambda b: (0, 0, 0)),
            pl.BlockSpec((1, _MID), lambda b: (0, 0)),
            pl.BlockSpec((_C, _MID), lambda b: (0, 0)),
            pl.BlockSpec((1, _C), lambda b: (0, 0)),
        ],
        out_specs=[
            pl.BlockSpec((1, _C, _P), lambda b: (b, 0, 0)),
            pl.BlockSpec((1, _C, _P), lambda b: (b, 0, 0)),
        ],
        out_shape=[
            jax.ShapeDtypeStruct((_B, _C, _P), jnp.float32),
            jax.ShapeDtypeStruct((_B, _C, _P), jnp.float32),
        ],
    )(xf, w1, b1, w2bd, b2, w3, b3)


def kernel(features, enabled, w1, b1, w2, b2, w3, b3):
    xf = features.reshape(_B, _C, _P)
    mask, scores = _run(
        xf,
        w1.reshape(_MID, _C),
        b1.reshape(1, _MID),
        _block_diag_w2(w2),
        b2.reshape(1, _MID),
        w3.reshape(_C, _MID),
        b3.reshape(1, _C),
    )
    mask4 = mask.reshape(_B, _C, _H, _W)
    scores4 = scores.reshape(_B, _C, _H, _W)
    en = jnp.asarray(enabled) != 0
    out_mask = jnp.where(en, mask4, jnp.float32(1.0))
    out_scores = jnp.where(en, scores4, jnp.float32(0.0))
    return out_mask, out_scores


# speedup vs baseline: 34.8013x; 1.0134x over previous
import jax
import jax.numpy as jnp
import numpy as np
from jax.experimental import pallas as pl
from jax.experimental.pallas import tpu as pltpu

_B, _C, _H, _W = 8, 384, 32, 32
_MID, _GROUPS = 96, 8
_P = _H * _W
_KEEP = 256
_PAD = 64
_INT_MIN = np.int32(-2147483648)
_SQRT1_2 = np.float32(1.0 / np.sqrt(2.0))


def _gelu(x):
    return 0.5 * x * (1.0 + jax.lax.erf(x * _SQRT1_2))


def _dot_t(a, b):
    return jax.lax.dot_general(a, b, (((1,), (1,)), ((), ())),
                               preferred_element_type=jnp.float32)


def _body(en_ref, x_ref, w1_ref, b1_ref, w2_ref, b2_ref, w3_ref, b3_ref,
          mask_ref, scores_ref):
    en = en_ref[0] != 0
    ones_row = jnp.ones((1, _P), jnp.float32)

    def _count(ind_bool):
        ind = jnp.where(ind_bool, 1.0, 0.0)
        return jax.lax.dot_general(ones_row, ind, (((1,), (0,)), ((), ())),
                                   preferred_element_type=jnp.float32)

    X = x_ref[0]
    H1 = _gelu(jax.lax.dot_general(X, w1_ref[...], (((0,), (1,)), ((), ())),
                                   preferred_element_type=jnp.float32)
               + b1_ref[...])
    zpad = jnp.zeros((_PAD, _MID), jnp.float32)
    H1p = jnp.concatenate([zpad, H1, zpad], axis=0)
    xrow = jax.lax.broadcasted_iota(jnp.int32, (_P, 1), 0) % _W
    acc = b2_ref[...] + jnp.zeros((_P, _MID), jnp.float32)
    for dy in (-1, 0, 1):
        for dx in (-1, 0, 1):
            s = dy * _W + dx
            sh = jax.lax.slice(H1p, (_PAD + s, 0), (_PAD + s + _P, _MID))
            if dx == 1:
                sh = jnp.where(xrow != (_W - 1), sh, 0.0)
            elif dx == -1:
                sh = jnp.where(xrow != 0, sh, 0.0)
            t = (dy + 1) * 3 + (dx + 1)
            acc = acc + _dot_t(sh, w2_ref[t])
    H2 = _gelu(acc)
    S = _dot_t(H2, w3_ref[...]) + b3_ref[...]
    scores_ref[0] = jnp.where(en, jnp.transpose(S, (1, 0)), 0.0)

    bits = jax.lax.bitcast_convert_type(S, jnp.int32)
    keys = jnp.where(bits >= 0, bits, bits ^ np.int32(0x7FFFFFFF))

    def step(i, T):
        cand_u = T | (np.int32(1) << (31 - i))
        cand_k = cand_u ^ _INT_MIN
        cnt = _count(keys < cand_k)
        return jnp.where(cnt < np.float32(_KEEP), cand_u, T)

    T = jax.lax.fori_loop(0, 32, step, jnp.zeros((1, _C), jnp.int32))
    Tk = T ^ _INT_MIN
    le = keys <= Tk
    cnt_le = _count(le)
    any_tie = jnp.sum(jnp.where(cnt_le > np.float32(_KEEP), 1, 0)) > 0

    def no_tie():
        return le.astype(jnp.float32)

    def tie_fix():
        lt = keys < Tk
        eq = keys == Tk
        g = jnp.sum(lt.astype(jnp.int32), axis=0, keepdims=True)
        c = eq.astype(jnp.int32)
        shift = 1
        while shift < _P:
            c = c + jnp.concatenate(
                [jnp.zeros((shift, _C), jnp.int32),
                 jax.lax.slice(c, (0, 0), (_P - shift, _C))], axis=0)
            shift *= 2
        return (lt | (eq & (c <= (_KEEP - g)))).astype(jnp.float32)

    maskT = jax.lax.cond(any_tie, tie_fix, no_tie)
    mask_ref[0] = jnp.where(en, jnp.transpose(maskT, (1, 0)), 1.0)


def _block_diag_w2(w2):
    cpg = _MID // _GROUPS
    take = jnp.take(w2, jnp.asarray(np.arange(_MID) % cpg), axis=1)
    gi = np.arange(_MID) // cpg
    gmask = jnp.asarray((gi[:, None] == gi[None, :]).astype(np.float32))
    full = take * gmask[:, :, None, None]
    return jnp.transpose(full, (2, 3, 0, 1)).reshape(9, _MID, _MID)


def _run(en, xf, w1, b1, w2bd, b2, w3, b3):
    return pl.pallas_call(
        _body,
        grid=(_B,),
        in_specs=[
            pl.BlockSpec(memory_space=pltpu.SMEM),
            pl.BlockSpec((1, _C, _P), lambda b: (b, 0, 0)),
            pl.BlockSpec((_MID, _C), lambda b: (0, 0)),
            pl.BlockSpec((1, _MID), lambda b: (0, 0)),
            pl.BlockSpec((9, _MID, _MID), lambda b: (0, 0, 0)),
            pl.BlockSpec((1, _MID), lambda b: (0, 0)),
            pl.BlockSpec((_C, _MID), lambda b: (0, 0)),
            pl.BlockSpec((1, _C), lambda b: (0, 0)),
        ],
        out_specs=[
            pl.BlockSpec((1, _C, _P), lambda b: (b, 0, 0)),
            pl.BlockSpec((1, _C, _P), lambda b: (b, 0, 0)),
        ],
        out_shape=[
            jax.ShapeDtypeStruct((_B, _C, _P), jnp.float32),
            jax.ShapeDtypeStruct((_B, _C, _P), jnp.float32),
        ],
    )(en, xf, w1, b1, w2bd, b2, w3, b3)


def kernel(features, enabled, w1, b1, w2, b2, w3, b3):
    xf = features.reshape(_B, _C, _P)
    mask, scores = _run(
        jnp.asarray(enabled, jnp.int32).reshape(1),
        xf,
        w1.reshape(_MID, _C),
        b1.reshape(1, _MID),
        _block_diag_w2(w2),
        b2.reshape(1, _MID),
        w3.reshape(_C, _MID),
        b3.reshape(1, _C),
    )
    return (mask.reshape(_B, _C, _H, _W), scores.reshape(_B, _C, _H, _W))
